# baseline (speedup 1.0000x reference)
import functools

import jax
import jax.numpy as jnp
from jax import lax
from jax.experimental import pallas as pl
from jax.experimental.pallas import tpu as pltpu
from jax.experimental.pallas import tpu_sc as plsc

V = 1000000
D = 64
B = 4096
L = 200
FEAT = 7 * D
LANES = 16

NC = 2
NS = 16
NW = NC * NS

VB = 128
NFULL = V // VB
TAIL = V - NFULL * VB
BPW = (NFULL + NW - 1) // NW

ROWS_PER_W = B // NW
C = 4
CHUNKS = ROWS_PER_W // C
S0, S1 = 104, 96


VBLK = 12800
QBLK = VBLK // 4
NBLK = (V + VBLK - 1) // VBLK
TROWS = NBLK * VBLK
PK = D // 2


def _detile_body(in_ref, out_ref):
    t = in_ref[...]
    blo = jax.lax.bitcast_convert_type(t[:PK, :], jnp.uint32)
    bhi = jax.lax.bitcast_convert_type(t[PK:, :], jnp.uint32)
    u = jax.lax.shift_right_logical(blo, jnp.uint32(16)) | (
        bhi & jnp.uint32(0xFFFF0000))
    ut = jnp.transpose(u)
    out_ref[...] = jnp.concatenate(
        [ut[i * QBLK:(i + 1) * QBLK] for i in range(4)], axis=1)


def _tc_detile(table):
    tabT = table.T
    out = pl.pallas_call(
        _detile_body,
        grid=(NBLK,),
        in_specs=[pl.BlockSpec((D, VBLK), lambda j: (0, j))],
        out_specs=pl.BlockSpec((QBLK, 128), lambda j: (j, 0)),
        out_shape=jax.ShapeDtypeStruct((NBLK * QBLK, 128), jnp.uint32),
    )(tabT)
    return out


def _permute_idx(x):
    b = x // VBLK
    r = x - b * VBLK
    q = r // QBLK
    p = r - q * QBLK
    return b * VBLK + 4 * p + q


def _sc_features(x, tab_lin):
    mesh = plsc.VectorSubcoreMesh(core_axis_name="c", subcore_axis_name="s")

    idx_t = pltpu.VMEM((C * L,), jnp.int32)
    mask_t = pltpu.VMEM((C * L + LANES,), jnp.float32)
    emb_t = pltpu.VMEM((C * L, PK), jnp.uint32)

    @functools.partial(
        pl.kernel,
        mesh=mesh,
        compiler_params=pltpu.CompilerParams(use_tc_tiling_on_sc=False),
        out_type=jax.ShapeDtypeStruct((B, FEAT), jnp.float32),
        scratch_types=[
            idx_t, idx_t,
            mask_t, mask_t,
            emb_t, emb_t,
            pltpu.VMEM((C, FEAT), jnp.float32),
            pltpu.SemaphoreType.DMA,
            pltpu.SemaphoreType.DMA,
        ],
    )
    def k(x_hbm, tab_hbm, out_hbm, idx_a, idx_b, mask_a, mask_b,
          emb_a, emb_b, feat_v, sem_a, sem_b):
        wid = lax.axis_index("s") * NC + lax.axis_index("c")
        row0 = wid * ROWS_PER_W

        def gather_pairs(idx_v, emb_v, sem):
            for r in range(C):
                o = r * L
                yield (tab_hbm.at[idx_v.at[pl.ds(o, S0)]],
                       emb_v.at[pl.ds(o, S0)], sem)
                yield (tab_hbm.at[idx_v.at[pl.ds(o + S0, S1)]],
                       emb_v.at[pl.ds(o + S0, S1)], sem)

        def issue(ci, idx_v, emb_v, sem):
            base = row0 + ci * C
            pltpu.sync_copy(x_hbm.at[pl.ds(base * L, C * L)], idx_v)
            for src, dst, s in gather_pairs(idx_v, emb_v, sem):
                pltpu.async_copy(src, dst, s)

        def wait_gathers(idx_v, emb_v, sem):
            for src, dst, s in gather_pairs(idx_v, emb_v, sem):
                pltpu.make_async_copy(src, dst, s).wait()

        zero = jnp.zeros((LANES,), jnp.float32)
        ninf = jnp.full((LANES,), -jnp.inf, jnp.float32)
        pinf = jnp.full((LANES,), jnp.inf, jnp.float32)
        inv_n = jnp.float32(1.0 / L)
        inv_nm1 = jnp.float32(1.0 / (L - 1))

        def consume(ci, idx_v, mask_v, emb_v):
            base = row0 + ci * C

            def mask_body(g, c):
                iv = idx_v[pl.ds(g * LANES, LANES)]
                mask_v[pl.ds(g * LANES, LANES)] = jnp.where(
                    iv != 0, jnp.float32(1.0), jnp.float32(0.0))
                return c

            lax.fori_loop(0, (C * L) // LANES, mask_body, 0)

            def unpacked(tt):
                u0 = emb_v[tt, pl.ds(0, LANES)]
                u1 = emb_v[tt, pl.ds(LANES, LANES)]
                himask = jnp.uint32(0xFFFF0000)
                return (
                    jax.lax.bitcast_convert_type(
                        jax.lax.shift_left(u0, jnp.uint32(16)), jnp.float32),
                    jax.lax.bitcast_convert_type(
                        jax.lax.shift_left(u1, jnp.uint32(16)), jnp.float32),
                    jax.lax.bitcast_convert_type(u0 & himask, jnp.float32),
                    jax.lax.bitcast_convert_type(u1 & himask, jnp.float32),
                )

            for r in range(C):
                o = r * L

                def tok_body(t, acc, o=o):
                    tt = o + t
                    m = jnp.full((LANES,), mask_v[pl.ds(tt, LANES)][0],
                                 jnp.float32)
                    vs = unpacked(tt)
                    out = []
                    for g in range(D // LANES):
                        s, q, mx, mn = acc[g]
                        vm = vs[g] * m
                        out.append((s + vm, q + vm * vm,
                                    jnp.maximum(mx, vm), jnp.minimum(mn, vm)))
                    return tuple(out)

                acc0 = tuple((zero, zero, ninf, pinf)
                             for _ in range(D // LANES))
                acc = lax.fori_loop(0, L, tok_body, acc0, unroll=2)

                for slot, t in ((0, 0), (1, L - 1), (2, L // 2)):
                    mt = jnp.full((LANES,), mask_v[pl.ds(o + t, LANES)][0],
                                  jnp.float32)
                    vs = unpacked(o + t)
                    for g in range(D // LANES):
                        feat_v[r, pl.ds(slot * D + g * LANES, LANES)] = (
                            vs[g] * mt)
                for g in range(D // LANES):
                    s, q, mx, mn = acc[g]
                    mean = s * inv_n
                    var = (q - s * mean) * inv_nm1
                    feat_v[r, pl.ds(3 * D + g * LANES, LANES)] = mean
                    feat_v[r, pl.ds(4 * D + g * LANES, LANES)] = mx
                    feat_v[r, pl.ds(5 * D + g * LANES, LANES)] = mn
                    feat_v[r, pl.ds(6 * D + g * LANES, LANES)] = var
            pltpu.sync_copy(feat_v, out_hbm.at[pl.ds(base, C)])

        issue(0, idx_a, emb_a, sem_a)

        def body(j, carry):
            issue(2 * j + 1, idx_b, emb_b, sem_b)
            wait_gathers(idx_a, emb_a, sem_a)
            consume(2 * j, idx_a, mask_a, emb_a)

            @pl.when(j < CHUNKS // 2 - 1)
            def _():
                issue(2 * j + 2, idx_a, emb_a, sem_a)

            wait_gathers(idx_b, emb_b, sem_b)
            consume(2 * j + 1, idx_b, mask_b, emb_b)
            return carry

        lax.fori_loop(0, CHUNKS // 2, body, 0)

    return k(x.reshape(B * L), tab_lin.reshape(TROWS, PK))


def _head_body(feat_ref, w_ref, b_ref, out_ref):
    f = feat_ref[...]
    w = w_ref[...]
    std = jnp.sqrt(jnp.maximum(f[:, 6 * D:], 0.0))
    out_ref[...] = (
        jax.lax.dot_general(f[:, :6 * D], w[:6 * D],
                            (((1,), (0,)), ((), ())),
                            preferred_element_type=jnp.float32)
        + jax.lax.dot_general(std, w[6 * D:],
                              (((1,), (0,)), ((), ())),
                              preferred_element_type=jnp.float32)
        + b_ref[...]
    )


def kernel(x, table, W, b):
    tab_lin = _tc_detile(table)
    features = _sc_features(_permute_idx(x), tab_lin)
    nclass = W.shape[1]
    return pl.pallas_call(
        _head_body,
        out_shape=jax.ShapeDtypeStruct((B, nclass), jnp.float32),
    )(features, W, b.reshape(1, nclass))


# speedup vs baseline: 1.1162x; 1.1162x over previous
import functools

import jax
import jax.numpy as jnp
from jax import lax
from jax.experimental import pallas as pl
from jax.experimental.pallas import tpu as pltpu
from jax.experimental.pallas import tpu_sc as plsc

V = 1000000
D = 64
B = 4096
L = 200
FEAT = 7 * D
LANES = 16

NC = 2
NS = 16
NW = NC * NS

VB = 128
NFULL = V // VB
TAIL = V - NFULL * VB
BPW = (NFULL + NW - 1) // NW

ROWS_PER_W = B // NW
C = 4
CHUNKS = ROWS_PER_W // C
S0, S1 = 104, 96


VBLK = 25600
HBLK = VBLK // 2
NBLK = (V + VBLK - 1) // VBLK
TROWS = NBLK * VBLK


def _detile_body(in_ref, out_ref):
    lo = jnp.transpose(in_ref[:, :HBLK])
    hi = jnp.transpose(in_ref[:, HBLK:])
    out_ref[...] = jnp.concatenate([lo, hi], axis=1)


def _tc_detile(table):
    tabT = table.T
    out = pl.pallas_call(
        _detile_body,
        grid=(NBLK,),
        in_specs=[pl.BlockSpec((D, VBLK), lambda j: (0, j))],
        out_specs=pl.BlockSpec((HBLK, 128), lambda j: (j, 0)),
        out_shape=jax.ShapeDtypeStruct((NBLK * HBLK, 128), jnp.float32),
    )(tabT)
    return out


def _permute_idx(x):
    b = x // VBLK
    r = x - b * VBLK
    q = r // HBLK
    p = r - q * HBLK
    return b * VBLK + 2 * p + q


def _sc_features(x, tab_lin):
    mesh = plsc.VectorSubcoreMesh(core_axis_name="c", subcore_axis_name="s")

    idx_t = pltpu.VMEM((C * L,), jnp.int32)
    mask_t = pltpu.VMEM((C * L + LANES,), jnp.float32)
    emb_t = pltpu.VMEM((C * L, D), jnp.float32)

    @functools.partial(
        pl.kernel,
        mesh=mesh,
        compiler_params=pltpu.CompilerParams(use_tc_tiling_on_sc=False),
        out_type=jax.ShapeDtypeStruct((B, FEAT), jnp.float32),
        scratch_types=[
            idx_t, idx_t,
            mask_t, mask_t,
            emb_t, emb_t,
            pltpu.VMEM((C, FEAT), jnp.float32),
            pltpu.SemaphoreType.DMA,
            pltpu.SemaphoreType.DMA,
        ],
    )
    def k(x_hbm, tab_hbm, out_hbm, idx_a, idx_b, mask_a, mask_b,
          emb_a, emb_b, feat_v, sem_a, sem_b):
        wid = lax.axis_index("s") * NC + lax.axis_index("c")
        row0 = wid * ROWS_PER_W

        def gather_pairs(idx_v, emb_v, sem):
            for r in range(C):
                o = r * L
                yield (tab_hbm.at[idx_v.at[pl.ds(o, S0)]],
                       emb_v.at[pl.ds(o, S0)], sem)
                yield (tab_hbm.at[idx_v.at[pl.ds(o + S0, S1)]],
                       emb_v.at[pl.ds(o + S0, S1)], sem)

        def issue(ci, idx_v, emb_v, sem):
            base = row0 + ci * C
            pltpu.sync_copy(x_hbm.at[pl.ds(base * L, C * L)], idx_v)
            for src, dst, s in gather_pairs(idx_v, emb_v, sem):
                pltpu.async_copy(src, dst, s)

        def wait_gathers(idx_v, emb_v, sem):
            for src, dst, s in gather_pairs(idx_v, emb_v, sem):
                pltpu.make_async_copy(src, dst, s).wait()

        zero = jnp.zeros((LANES,), jnp.float32)
        ninf = jnp.full((LANES,), -jnp.inf, jnp.float32)
        pinf = jnp.full((LANES,), jnp.inf, jnp.float32)
        inv_n = jnp.float32(1.0 / L)
        inv_nm1 = jnp.float32(1.0 / (L - 1))

        def consume(ci, idx_v, mask_v, emb_v):
            base = row0 + ci * C

            def mask_body(g, c):
                iv = idx_v[pl.ds(g * LANES, LANES)]
                mask_v[pl.ds(g * LANES, LANES)] = jnp.where(
                    iv != 0, jnp.float32(1.0), jnp.float32(0.0))
                return c

            lax.fori_loop(0, (C * L) // LANES, mask_body, 0)

            for r in range(C):
                o = r * L

                def tok_body(t, acc, o=o):
                    tt = o + t
                    m = jnp.full((LANES,), mask_v[pl.ds(tt, LANES)][0],
                                 jnp.float32)
                    out = []
                    for g in range(D // LANES):
                        s, q, mx, mn = acc[g]
                        v = emb_v[tt, pl.ds(g * LANES, LANES)]
                        vm = v * m
                        out.append((s + vm, q + vm * vm,
                                    jnp.maximum(mx, vm), jnp.minimum(mn, vm)))
                    return tuple(out)

                acc0 = tuple((zero, zero, ninf, pinf)
                             for _ in range(D // LANES))
                acc = lax.fori_loop(0, L, tok_body, acc0, unroll=2)

                for slot, t in ((0, 0), (1, L - 1), (2, L // 2)):
                    mt = jnp.full((LANES,), mask_v[pl.ds(o + t, LANES)][0],
                                  jnp.float32)
                    for g in range(D // LANES):
                        feat_v[r, pl.ds(slot * D + g * LANES, LANES)] = (
                            emb_v[o + t, pl.ds(g * LANES, LANES)] * mt)
                for g in range(D // LANES):
                    s, q, mx, mn = acc[g]
                    mean = s * inv_n
                    var = (q - s * mean) * inv_nm1
                    feat_v[r, pl.ds(3 * D + g * LANES, LANES)] = mean
                    feat_v[r, pl.ds(4 * D + g * LANES, LANES)] = mx
                    feat_v[r, pl.ds(5 * D + g * LANES, LANES)] = mn
                    feat_v[r, pl.ds(6 * D + g * LANES, LANES)] = var
            pltpu.sync_copy(feat_v, out_hbm.at[pl.ds(base, C)])

        issue(0, idx_a, emb_a, sem_a)

        def body(j, carry):
            issue(2 * j + 1, idx_b, emb_b, sem_b)
            wait_gathers(idx_a, emb_a, sem_a)
            consume(2 * j, idx_a, mask_a, emb_a)

            @pl.when(j < CHUNKS // 2 - 1)
            def _():
                issue(2 * j + 2, idx_a, emb_a, sem_a)

            wait_gathers(idx_b, emb_b, sem_b)
            consume(2 * j + 1, idx_b, mask_b, emb_b)
            return carry

        lax.fori_loop(0, CHUNKS // 2, body, 0)

    return k(x.reshape(B * L), tab_lin.reshape(TROWS, D))


def _head_body(feat_ref, w_ref, b_ref, out_ref):
    f = feat_ref[...]
    w = w_ref[...]
    std = jnp.sqrt(jnp.maximum(f[:, 6 * D:], 0.0))
    out_ref[...] = (
        jax.lax.dot_general(f[:, :6 * D], w[:6 * D],
                            (((1,), (0,)), ((), ())),
                            preferred_element_type=jnp.float32)
        + jax.lax.dot_general(std, w[6 * D:],
                              (((1,), (0,)), ((), ())),
                              preferred_element_type=jnp.float32)
        + b_ref[...]
    )


def kernel(x, table, W, b):
    tab_lin = _tc_detile(table)
    features = _sc_features(_permute_idx(x), tab_lin)
    nclass = W.shape[1]
    return pl.pallas_call(
        _head_body,
        out_shape=jax.ShapeDtypeStruct((B, nclass), jnp.float32),
    )(features, W, b.reshape(1, nclass))


# speedup vs baseline: 1.1221x; 1.0052x over previous
import functools

import jax
import jax.numpy as jnp
from jax import lax
from jax.experimental import pallas as pl
from jax.experimental.pallas import tpu as pltpu
from jax.experimental.pallas import tpu_sc as plsc

V = 1000000
D = 64
B = 4096
L = 200
FEAT = 7 * D
LANES = 16

NC = 2
NS = 16
NW = NC * NS

VB = 128
NFULL = V // VB
TAIL = V - NFULL * VB
BPW = (NFULL + NW - 1) // NW

ROWS_PER_W = B // NW
C = 4
CHUNKS = ROWS_PER_W // C
S0, S1 = 104, 96


VBLK = 32000
HBLK = VBLK // 2
NBLK = (V + VBLK - 1) // VBLK
TROWS = NBLK * VBLK


def _detile_body(in_ref, out_ref):
    lo = jnp.transpose(in_ref[:, :HBLK])
    hi = jnp.transpose(in_ref[:, HBLK:])
    out_ref[...] = jnp.concatenate([lo, hi], axis=1)


def _tc_detile(table):
    tabT = table.T
    out = pl.pallas_call(
        _detile_body,
        grid=(NBLK,),
        in_specs=[pl.BlockSpec((D, VBLK), lambda j: (0, j))],
        out_specs=pl.BlockSpec((HBLK, 128), lambda j: (j, 0)),
        out_shape=jax.ShapeDtypeStruct((NBLK * HBLK, 128), jnp.float32),
    )(tabT)
    return out


def _permute_idx(x):
    b = x // VBLK
    r = x - b * VBLK
    q = r // HBLK
    p = r - q * HBLK
    return b * VBLK + 2 * p + q


def _sc_features(x, tab_lin):
    mesh = plsc.VectorSubcoreMesh(core_axis_name="c", subcore_axis_name="s")

    idx_t = pltpu.VMEM((C * L,), jnp.int32)
    mask_t = pltpu.VMEM((C * L + LANES,), jnp.float32)
    emb_t = pltpu.VMEM((C * L, D), jnp.float32)

    @functools.partial(
        pl.kernel,
        mesh=mesh,
        compiler_params=pltpu.CompilerParams(use_tc_tiling_on_sc=False),
        out_type=jax.ShapeDtypeStruct((B, FEAT), jnp.float32),
        scratch_types=[
            idx_t, idx_t,
            mask_t, mask_t,
            emb_t, emb_t,
            pltpu.VMEM((C, FEAT), jnp.float32),
            pltpu.SemaphoreType.DMA,
            pltpu.SemaphoreType.DMA,
        ],
    )
    def k(x_hbm, tab_hbm, out_hbm, idx_a, idx_b, mask_a, mask_b,
          emb_a, emb_b, feat_v, sem_a, sem_b):
        wid = lax.axis_index("s") * NC + lax.axis_index("c")
        row0 = wid * ROWS_PER_W

        def gather_pairs(idx_v, emb_v, sem):
            for r in range(C):
                o = r * L
                yield (tab_hbm.at[idx_v.at[pl.ds(o, S0)]],
                       emb_v.at[pl.ds(o, S0)], sem)
                yield (tab_hbm.at[idx_v.at[pl.ds(o + S0, S1)]],
                       emb_v.at[pl.ds(o + S0, S1)], sem)

        def issue(ci, idx_v, emb_v, sem):
            base = row0 + ci * C
            pltpu.sync_copy(x_hbm.at[pl.ds(base * L, C * L)], idx_v)
            for src, dst, s in gather_pairs(idx_v, emb_v, sem):
                pltpu.async_copy(src, dst, s)

        def wait_gathers(idx_v, emb_v, sem):
            for src, dst, s in gather_pairs(idx_v, emb_v, sem):
                pltpu.make_async_copy(src, dst, s).wait()

        zero = jnp.zeros((LANES,), jnp.float32)
        ninf = jnp.full((LANES,), -jnp.inf, jnp.float32)
        pinf = jnp.full((LANES,), jnp.inf, jnp.float32)
        inv_n = jnp.float32(1.0 / L)
        inv_nm1 = jnp.float32(1.0 / (L - 1))

        def consume(ci, idx_v, mask_v, emb_v):
            base = row0 + ci * C

            def mask_body(g, c):
                iv = idx_v[pl.ds(g * LANES, LANES)]
                mask_v[pl.ds(g * LANES, LANES)] = jnp.where(
                    iv != 0, jnp.float32(1.0), jnp.float32(0.0))
                return c

            lax.fori_loop(0, (C * L) // LANES, mask_body, 0)

            for r in range(C):
                o = r * L

                def tok_body(t, acc, o=o):
                    tt = o + t
                    m = jnp.full((LANES,), mask_v[pl.ds(tt, LANES)][0],
                                 jnp.float32)
                    out = []
                    for g in range(D // LANES):
                        s, q, mx, mn = acc[g]
                        v = emb_v[tt, pl.ds(g * LANES, LANES)]
                        vm = v * m
                        out.append((s + vm, q + vm * vm,
                                    jnp.maximum(mx, vm), jnp.minimum(mn, vm)))
                    return tuple(out)

                acc0 = tuple((zero, zero, ninf, pinf)
                             for _ in range(D // LANES))
                acc = lax.fori_loop(0, L, tok_body, acc0, unroll=2)

                for slot, t in ((0, 0), (1, L - 1), (2, L // 2)):
                    mt = jnp.full((LANES,), mask_v[pl.ds(o + t, LANES)][0],
                                  jnp.float32)
                    for g in range(D // LANES):
                        feat_v[r, pl.ds(slot * D + g * LANES, LANES)] = (
                            emb_v[o + t, pl.ds(g * LANES, LANES)] * mt)
                for g in range(D // LANES):
                    s, q, mx, mn = acc[g]
                    mean = s * inv_n
                    var = (q - s * mean) * inv_nm1
                    feat_v[r, pl.ds(3 * D + g * LANES, LANES)] = mean
                    feat_v[r, pl.ds(4 * D + g * LANES, LANES)] = mx
                    feat_v[r, pl.ds(5 * D + g * LANES, LANES)] = mn
                    feat_v[r, pl.ds(6 * D + g * LANES, LANES)] = var
            pltpu.sync_copy(feat_v, out_hbm.at[pl.ds(base, C)])

        issue(0, idx_a, emb_a, sem_a)

        def body(j, carry):
            issue(2 * j + 1, idx_b, emb_b, sem_b)
            wait_gathers(idx_a, emb_a, sem_a)
            consume(2 * j, idx_a, mask_a, emb_a)

            @pl.when(j < CHUNKS // 2 - 1)
            def _():
                issue(2 * j + 2, idx_a, emb_a, sem_a)

            wait_gathers(idx_b, emb_b, sem_b)
            consume(2 * j + 1, idx_b, mask_b, emb_b)
            return carry

        lax.fori_loop(0, CHUNKS // 2, body, 0)

    return k(x.reshape(B * L), tab_lin.reshape(TROWS, D))


def _head_body(feat_ref, w_ref, b_ref, out_ref):
    f = feat_ref[...]
    w = w_ref[...]
    std = jnp.sqrt(jnp.maximum(f[:, 6 * D:], 0.0))
    out_ref[...] = (
        jax.lax.dot_general(f[:, :6 * D], w[:6 * D],
                            (((1,), (0,)), ((), ())),
                            preferred_element_type=jnp.float32)
        + jax.lax.dot_general(std, w[6 * D:],
                              (((1,), (0,)), ((), ())),
                              preferred_element_type=jnp.float32)
        + b_ref[...]
    )


def kernel(x, table, W, b):
    tab_lin = _tc_detile(table)
    features = _sc_features(_permute_idx(x), tab_lin)
    nclass = W.shape[1]
    return pl.pallas_call(
        _head_body,
        out_shape=jax.ShapeDtypeStruct((B, nclass), jnp.float32),
    )(features, W, b.reshape(1, nclass))


# speedup vs baseline: 1.1226x; 1.0005x over previous
import functools

import jax
import jax.numpy as jnp
from jax import lax
from jax.experimental import pallas as pl
from jax.experimental.pallas import tpu as pltpu
from jax.experimental.pallas import tpu_sc as plsc

V = 1000000
D = 64
B = 4096
L = 200
FEAT = 7 * D
LANES = 16

NC = 2
NS = 16
NW = NC * NS

ROWS_PER_W = B // NW
C = 4
CHUNKS = ROWS_PER_W // C
S0, S1 = 104, 96


VBLK = 32000
HBLK = VBLK // 2
NBLK = (V + VBLK - 1) // VBLK
TROWS = NBLK * VBLK


def _detile_body(in_ref, out_ref):
    lo = jnp.transpose(in_ref[:, :HBLK])
    hi = jnp.transpose(in_ref[:, HBLK:])
    out_ref[...] = jnp.concatenate([lo, hi], axis=1)


def _tc_detile(table):
    tabT = table.T
    out = pl.pallas_call(
        _detile_body,
        grid=(NBLK,),
        in_specs=[pl.BlockSpec((D, VBLK), lambda j: (0, j))],
        out_specs=pl.BlockSpec((HBLK, 128), lambda j: (j, 0)),
        out_shape=jax.ShapeDtypeStruct((NBLK * HBLK, 128), jnp.float32),
    )(tabT)
    return out


def _permute_idx(x):
    b = x // VBLK
    r = x - b * VBLK
    q = r // HBLK
    p = r - q * HBLK
    return b * VBLK + 2 * p + q


def _sc_features(x, tab_lin):
    mesh = plsc.VectorSubcoreMesh(core_axis_name="c", subcore_axis_name="s")

    idx_t = pltpu.VMEM((C * L,), jnp.int32)
    mask_t = pltpu.VMEM((C * L + LANES,), jnp.float32)
    emb_t = pltpu.VMEM((C * L, D), jnp.float32)

    @functools.partial(
        pl.kernel,
        mesh=mesh,
        compiler_params=pltpu.CompilerParams(use_tc_tiling_on_sc=False),
        out_type=jax.ShapeDtypeStruct((B, FEAT), jnp.float32),
        scratch_types=[
            idx_t, idx_t,
            mask_t, mask_t,
            emb_t, emb_t,
            pltpu.VMEM((C, FEAT), jnp.float32),
            pltpu.SemaphoreType.DMA,
            pltpu.SemaphoreType.DMA,
        ],
    )
    def k(x_hbm, tab_hbm, out_hbm, idx_a, idx_b, mask_a, mask_b,
          emb_a, emb_b, feat_v, sem_a, sem_b):
        wid = lax.axis_index("s") * NC + lax.axis_index("c")
        row0 = wid * ROWS_PER_W

        def gather_pairs(idx_v, emb_v, sem):
            for r in range(C):
                o = r * L
                yield (tab_hbm.at[idx_v.at[pl.ds(o, S0)]],
                       emb_v.at[pl.ds(o, S0)], sem)
                yield (tab_hbm.at[idx_v.at[pl.ds(o + S0, S1)]],
                       emb_v.at[pl.ds(o + S0, S1)], sem)

        def issue(ci, idx_v, emb_v, sem):
            base = row0 + ci * C
            pltpu.sync_copy(x_hbm.at[pl.ds(base * L, C * L)], idx_v)
            for src, dst, s in gather_pairs(idx_v, emb_v, sem):
                pltpu.async_copy(src, dst, s)

        def wait_gathers(idx_v, emb_v, sem):
            for src, dst, s in gather_pairs(idx_v, emb_v, sem):
                pltpu.make_async_copy(src, dst, s).wait()

        zero = jnp.zeros((LANES,), jnp.float32)
        ninf = jnp.full((LANES,), -jnp.inf, jnp.float32)
        pinf = jnp.full((LANES,), jnp.inf, jnp.float32)
        inv_n = jnp.float32(1.0 / L)
        inv_nm1 = jnp.float32(1.0 / (L - 1))

        def consume(ci, idx_v, mask_v, emb_v):
            base = row0 + ci * C

            def mask_body(g, c):
                iv = idx_v[pl.ds(g * LANES, LANES)]
                mask_v[pl.ds(g * LANES, LANES)] = jnp.where(
                    iv != 0, jnp.float32(1.0), jnp.float32(0.0))
                return c

            lax.fori_loop(0, (C * L) // LANES, mask_body, 0)

            for r in range(C):
                o = r * L

                def tok_body(t, acc, o=o):
                    tt = o + t
                    m = jnp.full((LANES,), mask_v[pl.ds(tt, LANES)][0],
                                 jnp.float32)
                    out = []
                    for g in range(D // LANES):
                        s, q, mx, mn = acc[g]
                        v = emb_v[tt, pl.ds(g * LANES, LANES)]
                        vm = v * m
                        out.append((s + vm, q + vm * vm,
                                    jnp.maximum(mx, vm), jnp.minimum(mn, vm)))
                    return tuple(out)

                acc0 = tuple((zero, zero, ninf, pinf)
                             for _ in range(D // LANES))
                acc = lax.fori_loop(0, L, tok_body, acc0, unroll=2)

                for slot, t in ((0, 0), (1, L - 1), (2, L // 2)):
                    mt = jnp.full((LANES,), mask_v[pl.ds(o + t, LANES)][0],
                                  jnp.float32)
                    for g in range(D // LANES):
                        feat_v[r, pl.ds(slot * D + g * LANES, LANES)] = (
                            emb_v[o + t, pl.ds(g * LANES, LANES)] * mt)
                for g in range(D // LANES):
                    s, q, mx, mn = acc[g]
                    mean = s * inv_n
                    var = (q - s * mean) * inv_nm1
                    feat_v[r, pl.ds(3 * D + g * LANES, LANES)] = mean
                    feat_v[r, pl.ds(4 * D + g * LANES, LANES)] = mx
                    feat_v[r, pl.ds(5 * D + g * LANES, LANES)] = mn
                    feat_v[r, pl.ds(6 * D + g * LANES, LANES)] = var
            pltpu.sync_copy(feat_v, out_hbm.at[pl.ds(base, C)])

        issue(0, idx_a, emb_a, sem_a)

        def body(j, carry):
            issue(2 * j + 1, idx_b, emb_b, sem_b)
            wait_gathers(idx_a, emb_a, sem_a)
            consume(2 * j, idx_a, mask_a, emb_a)

            @pl.when(j < CHUNKS // 2 - 1)
            def _():
                issue(2 * j + 2, idx_a, emb_a, sem_a)

            wait_gathers(idx_b, emb_b, sem_b)
            consume(2 * j + 1, idx_b, mask_b, emb_b)
            return carry

        lax.fori_loop(0, CHUNKS // 2, body, 0)

    return k(x.reshape(B * L), tab_lin.reshape(TROWS, D))


def _head_body(feat_ref, w_ref, b_ref, out_ref):
    f = feat_ref[...]
    w = w_ref[...]
    std = jnp.sqrt(jnp.maximum(f[:, 6 * D:], 0.0))
    out_ref[...] = (
        jax.lax.dot_general(f[:, :6 * D], w[:6 * D],
                            (((1,), (0,)), ((), ())),
                            preferred_element_type=jnp.float32)
        + jax.lax.dot_general(std, w[6 * D:],
                              (((1,), (0,)), ((), ())),
                              preferred_element_type=jnp.float32)
        + b_ref[...]
    )


def kernel(x, table, W, b):
    tab_lin = _tc_detile(table)
    features = _sc_features(_permute_idx(x), tab_lin)
    nclass = W.shape[1]
    return pl.pallas_call(
        _head_body,
        out_shape=jax.ShapeDtypeStruct((B, nclass), jnp.float32),
    )(features, W, b.reshape(1, nclass))
